# baseline (speedup 1.0000x reference)
import functools

import jax
import jax.numpy as jnp
from jax import lax
from jax.experimental import pallas as pl
from jax.experimental.pallas import tpu as pltpu
from jax.experimental.pallas import tpu_sc as plsc

D = 64
K = 20
CH = 32
LANES = 16
IDX_MAX = 128


@functools.lru_cache(maxsize=None)
def _make_sc_kernel(B, V):
    info = plsc.get_sparse_core_info()
    NC, NS = info.num_cores, info.num_subcores
    NW = NC * NS
    bpw = B // NW
    nch = bpw // CH
    f32 = jnp.float32
    mesh = plsc.VectorSubcoreMesh(core_axis_name="c", subcore_axis_name="s")

    @functools.partial(
        pl.kernel,
        out_type=(
            jax.ShapeDtypeStruct((B, LANES), f32),
            jax.ShapeDtypeStruct((B, LANES), f32),
        ),
        mesh=mesh,
        compiler_params=pltpu.CompilerParams(use_tc_tiling_on_sc=False),
        scratch_types=[
            pltpu.VMEM((2, CH), jnp.int32),
            pltpu.VMEM((2, CH), jnp.int32),
            pltpu.VMEM((2, CH * K), jnp.int32),
            pltpu.VMEM((2, CH, D), f32),
            pltpu.VMEM((2, CH, D), f32),
            pltpu.VMEM((2, CH * K, D), f32),
            pltpu.VMEM((bpw, LANES), f32),
            pltpu.VMEM((bpw, LANES), f32),
            pltpu.SemaphoreType.DMA,
            pltpu.SemaphoreType.DMA,
            pltpu.SemaphoreType.DMA,
            pltpu.SemaphoreType.DMA,
        ],
    )
    def sc_kernel(tgt_hbm, ctx_hbm, negf_hbm, u_hbm, v_hbm,
                  pos_hbm, negp_hbm,
                  tgt_i, ctx_i, neg_i, urows, vrows, nrows,
                  pos_pv, neg_pv,
                  isem0, isem1, rsem0, rsem1):
        wid = lax.axis_index("s") * NC + lax.axis_index("c")
        base = wid * bpw
        isems = (isem0, isem1)
        rsems = (rsem0, rsem1)

        def fire_idx(c):
            s = c % 2
            b = base + c * CH
            return [
                pltpu.async_copy(tgt_hbm.at[pl.ds(b, CH)], tgt_i.at[s],
                                 isems[s]),
                pltpu.async_copy(ctx_hbm.at[pl.ds(b, CH)], ctx_i.at[s],
                                 isems[s]),
                pltpu.async_copy(negf_hbm.at[pl.ds(b * K, CH * K)],
                                 neg_i.at[s], isems[s]),
            ]

        def fire_rows(c):
            s = c % 2
            hs = [
                pltpu.async_copy(u_hbm.at[tgt_i.at[s]], urows.at[s],
                                 rsems[s]),
                pltpu.async_copy(v_hbm.at[ctx_i.at[s]], vrows.at[s],
                                 rsems[s]),
            ]
            for j in range(CH * K // IDX_MAX):
                hs.append(pltpu.async_copy(
                    v_hbm.at[neg_i.at[s, pl.ds(j * IDX_MAX, IDX_MAX)]],
                    nrows.at[s, pl.ds(j * IDX_MAX, IDX_MAX)],
                    rsems[s]))
            return hs

        def compute(c):
            s = c % 2

            def body(i, carry):
                rb = i * K
                acc = [nrows[s, rb, pl.ds(16 * d, 16)] for d in range(4)]
                for k in range(1, K):
                    for d in range(4):
                        acc[d] = acc[d] + nrows[s, rb + k, pl.ds(16 * d, 16)]
                pv = None
                nv = None
                for d in range(4):
                    u = urows[s, i, pl.ds(16 * d, 16)]
                    v = vrows[s, i, pl.ds(16 * d, 16)]
                    pv = u * v if pv is None else pv + u * v
                    nv = u * acc[d] if nv is None else nv + u * acc[d]
                row = c * CH + i
                pos_pv[row] = pv
                neg_pv[row] = nv
                return carry

            lax.fori_loop(0, CH, body, 0)

        ih = {0: fire_idx(0)}
        if nch > 1:
            ih[1] = fire_idx(1)
        rh = {}
        for h in ih.pop(0):
            h.wait()
        rh[0] = fire_rows(0)
        for c in range(nch):
            if c + 1 < nch:
                for h in ih.pop(c + 1):
                    h.wait()
                rh[c + 1] = fire_rows(c + 1)
            for h in rh.pop(c):
                h.wait()
            if c + 2 < nch:
                ih[c + 2] = fire_idx(c + 2)
            compute(c)

        pltpu.sync_copy(pos_pv, pos_hbm.at[pl.ds(base, bpw)])
        pltpu.sync_copy(neg_pv, negp_hbm.at[pl.ds(base, bpw)])

    return sc_kernel


def _finish_body(pos_ref, neg_ref, out_ref):
    p = jnp.sum(pos_ref[...], axis=1)
    n = jnp.sum(neg_ref[...], axis=1)
    loss = jax.nn.log_sigmoid(p) + jax.nn.log_sigmoid(-n)
    total = -jnp.sum(loss) * (1.0 / p.shape[0])
    out_ref[...] = total * jnp.ones((1, 1), jnp.float32)


@jax.jit
def kernel(target, context, neg, u_emb, v_emb):
    B, Kn = neg.shape
    V, Dn = u_emb.shape
    assert Kn == K and Dn == D
    tgt = target.astype(jnp.int32)
    ctx = context.astype(jnp.int32)
    negf = neg.astype(jnp.int32).reshape(B * K)
    pos_pv, neg_pv = _make_sc_kernel(B, V)(tgt, ctx, negf, u_emb, v_emb)
    out = pl.pallas_call(
        _finish_body,
        out_shape=jax.ShapeDtypeStruct((1, 1), jnp.float32),
    )(pos_pv, neg_pv)
    return out[0, 0]
